# baseline (speedup 1.0000x reference)
import jax
import jax.numpy as jnp
from jax.experimental import pallas as pl


def _gather_mm(x0_ref, x1_ref, m0_ref, m1_ref, o_ref):
    a = jnp.dot(x0_ref[...], m0_ref[...], preferred_element_type=jnp.float32)
    b = jnp.dot(x1_ref[...], m1_ref[...], preferred_element_type=jnp.float32)
    o_ref[...] = a + b


def kernel(x, inds):
    n_rows, n_cols = x.shape
    k = inds.shape[0]
    half = n_cols // 2
    m = (inds[None, :] == jnp.arange(n_cols, dtype=inds.dtype)[:, None])
    m = m.astype(x.dtype)
    x0, x1 = x[:, :half], x[:, half:]
    m0, m1 = m[:half], m[half:]

    block_rows = 2048
    grid = (n_rows // block_rows,)
    return pl.pallas_call(
        _gather_mm,
        grid=grid,
        in_specs=[
            pl.BlockSpec((block_rows, half), lambda i: (i, 0)),
            pl.BlockSpec((block_rows, half), lambda i: (i, 0)),
            pl.BlockSpec((half, k), lambda i: (0, 0)),
            pl.BlockSpec((half, k), lambda i: (0, 0)),
        ],
        out_specs=pl.BlockSpec((block_rows, k), lambda i: (i, 0)),
        out_shape=jax.ShapeDtypeStruct((n_rows, k), x.dtype),
    )(x0, x1, m0, m1)


# speedup vs baseline: 2.3332x; 2.3332x over previous
import jax
import jax.numpy as jnp
from jax.experimental import pallas as pl
from jax.experimental.pallas import tpu as pltpu

_NBUF = 4
_BLK = 1024


def _gather_mm(x_hbm, m_ref, o_ref, bufs, sems):
    n_rows = x_hbm.shape[0]
    n_blocks = n_rows // _BLK

    def start(i):
        pltpu.make_async_copy(
            x_hbm.at[pl.ds(i * _BLK, _BLK), :],
            bufs.at[i % _NBUF],
            sems.at[i % _NBUF],
        ).start()

    for i in range(_NBUF):
        start(i)
    for i in range(n_blocks):
        b = i % _NBUF
        pltpu.make_async_copy(
            x_hbm.at[pl.ds(i * _BLK, _BLK), :], bufs.at[b], sems.at[b]
        ).wait()
        o_ref[pl.ds(i * _BLK, _BLK), :] = jnp.dot(
            bufs[b], m_ref[...], preferred_element_type=jnp.float32
        )
        if i + _NBUF < n_blocks:
            start(i + _NBUF)


def kernel(x, inds):
    n_rows, n_cols = x.shape
    k = inds.shape[0]
    m = (inds[None, :] == jnp.arange(n_cols, dtype=inds.dtype)[:, None])
    m = m.astype(x.dtype)

    return pl.pallas_call(
        _gather_mm,
        in_specs=[
            pl.BlockSpec(memory_space=pl.ANY),
            pl.BlockSpec((n_cols, k), lambda: (0, 0)),
        ],
        out_specs=pl.BlockSpec((n_rows, k), lambda: (0, 0)),
        out_shape=jax.ShapeDtypeStruct((n_rows, k), x.dtype),
        scratch_shapes=[
            pltpu.VMEM((_NBUF, _BLK, n_cols), x.dtype),
            pltpu.SemaphoreType.DMA((_NBUF,)),
        ],
    )(x, m)
